# baseline (speedup 1.0000x reference)
import functools

import jax
import jax.numpy as jnp
from jax import lax
from jax.experimental import pallas as pl
from jax.experimental.pallas import tpu as pltpu
from jax.experimental.pallas import tpu_sc as plsc

_L = 16
_C = 4096
_R = 4 * 4096
_NW = 32
_RB = 8
_PH = 2
_PR = _RB // _PH
_ROWS_PER_W = _R // _NW
_N_CHUNKS = _ROWS_PER_W // _RB
_J = _C // _L


def _permute_body(y_hbm, perm_hbm, out_hbm, perm_v,
                  in0, in1, out0, out1, si0, si1, so0, so1):
    wid = lax.axis_index("s") * 2 + lax.axis_index("c")
    row0 = wid * _ROWS_PER_W
    pltpu.sync_copy(perm_hbm, perm_v)

    rvecs = [jnp.full((_L,), r, jnp.int32) for r in range(_RB)]

    ins = (in0, in1)
    outs = (out0, out1)
    sis = (si0, si1)
    sos = (so0, so1)

    def in_slice(c):
        return y_hbm.at[pl.ds(row0 + c * _RB, _RB)]

    def ph_slice(c, h):
        return out_hbm.at[pl.ds(row0 + c * _RB + h * _PR, _PR)]

    pltpu.async_copy(in_slice(0), in0, si0)
    pltpu.async_copy(in_slice(1), in1, si1)

    @pl.loop(0, _N_CHUNKS, step=2)
    def chunk_loop(c0):
        for b in range(2):
            c = c0 + b
            in_v, si = ins[b], sis[b]
            pltpu.make_async_copy(in_slice(c), in_v, si).wait()
            for h in range(_PH):
                out_v, so = outs[h], sos[h]
                @pl.when(c >= 1)
                def _():
                    pltpu.make_async_copy(out_v, ph_slice(c - 1, h), so).wait()

                @plsc.parallel_loop(0, _J, unroll=4)
                def gather(j):
                    pj = perm_v[pl.ds(j * _L, _L)]
                    for r in range(_PR):
                        out_v[r, pl.ds(j * _L, _L)] = plsc.load_gather(
                            in_v, [rvecs[h * _PR + r], pj])

                pltpu.async_copy(out_v, ph_slice(c, h), so)

            @pl.when(c + 2 < _N_CHUNKS)
            def _():
                pltpu.async_copy(in_slice(c + 2), in_v, si)

    pltpu.make_async_copy(out0, ph_slice(_N_CHUNKS - 1, 0), so0).wait()
    pltpu.make_async_copy(out1, ph_slice(_N_CHUNKS - 1, 1), so1).wait()


@jax.jit
def _permute(y2, perm_i32):
    mesh = plsc.VectorSubcoreMesh(core_axis_name="c", subcore_axis_name="s")
    f = functools.partial(
        pl.kernel,
        mesh=mesh,
        out_type=jax.ShapeDtypeStruct((_R, _C), jnp.float32),
        scratch_types=(
            [pltpu.VMEM((_C,), jnp.int32)]
            + [pltpu.VMEM((_RB, _C), jnp.float32)] * 2
            + [pltpu.VMEM((_PR, _C), jnp.float32)] * 2
            + [pltpu.SemaphoreType.DMA] * 4
        ),
        compiler_params=pltpu.CompilerParams(needs_layout_passes=False),
    )(_permute_body)
    return f(y2, perm_i32)


def kernel(y, perm):
    out = _permute(y.reshape(_R, _C), perm.astype(jnp.int32))
    return out.reshape(y.shape)
